# Initial kernel scaffold; baseline (speedup 1.0000x reference)
import jax
import jax.numpy as jnp
from jax.experimental import pallas as pl


def kernel(x, edge_index, W, b, gamma, beta):
    raise NotImplementedError("write your pallas kernel here")


# speedup vs baseline: 23.9181x; 23.9181x over previous
import functools

import jax
import jax.numpy as jnp
from jax import lax
from jax.experimental import pallas as pl
from jax.experimental.pallas import tpu as pltpu
from jax.experimental.pallas import tpu_sc as plsc

N = 10000
E = 320000
D = 128
EPS = 1e-08

NC = 2
NS = 16
NW = NC * NS

EPT = E // NW
CH = 80
NCHUNK = EPT // CH
ZR = 16
NZB = N // ZR
ZPT = (NZB + NS - 1) // NS


def _mesh():
    return plsc.VectorSubcoreMesh(
        core_axis_name="c", subcore_axis_name="s",
        num_cores=NC, num_subcores=NS)


def _deg_body(dst_hbm, degp_hbm, idx_v, deg_v):
    c = lax.axis_index("c")
    s = lax.axis_index("s")
    wid = c * NS + s
    pltpu.sync_copy(dst_hbm.at[pl.ds(wid * EPT, EPT)], idx_v)

    zeros16 = jnp.zeros((16,), jnp.int32)

    def zbody(i, _):
        deg_v[pl.ds(i * 16, 16)] = zeros16
        return 0

    lax.fori_loop(0, N // 16, zbody, 0)

    ones16 = jnp.ones((16,), jnp.int32)

    def body(i, _):
        idx = idx_v[pl.ds(i * 16, 16)]
        plsc.addupdate_scatter(deg_v, [idx], ones16)
        return 0

    lax.fori_loop(0, EPT // 16, body, 0)
    pltpu.sync_copy(deg_v, degp_hbm.at[wid])


def _deg_call(dst):
    return pl.kernel(
        _deg_body,
        out_type=jax.ShapeDtypeStruct((NW, N), jnp.int32),
        mesh=_mesh(),
        scratch_types=[
            pltpu.VMEM((EPT,), jnp.int32),
            pltpu.VMEM((N,), jnp.int32),
        ],
        compiler_params=pltpu.CompilerParams(needs_layout_passes=False),
    )(dst)


def _scat_body(hs_hbm, src_hbm, dst_hbm, out_hbm,
               acc_sh, srcv, dstv, rows, zbuf, sem):
    c = lax.axis_index("c")
    s = lax.axis_index("s")
    wid = c * NS + s

    zeros16f = jnp.zeros((16,), jnp.float32)

    def zinit(i, _):
        zbuf[i // (D // 16), pl.ds((i % (D // 16)) * 16, 16)] = zeros16f
        return 0

    lax.fori_loop(0, ZR * (D // 16), zinit, 0)

    def zbody(t, _):
        i = t * NS + s

        @pl.when(i < NZB)
        def _():
            pltpu.sync_copy(zbuf, acc_sh.at[pl.ds(i * ZR, ZR)])
        return 0

    lax.fori_loop(0, ZPT, zbody, 0)
    plsc.subcore_barrier()

    pltpu.sync_copy(src_hbm.at[wid], srcv)
    pltpu.sync_copy(dst_hbm.at[wid], dstv)

    def body(j, _):
        pltpu.async_copy(hs_hbm.at[srcv.at[j]], rows, sem).wait()
        pltpu.sync_copy(rows, acc_sh.at[dstv.at[j]], add=True)
        return 0

    lax.fori_loop(0, NCHUNK, body, 0)
    plsc.subcore_barrier()

    def wbody(t, _):
        i = t * NS + s

        @pl.when(i < NZB)
        def _():
            pltpu.sync_copy(acc_sh.at[pl.ds(i * ZR, ZR)],
                            out_hbm.at[c].at[pl.ds(i * ZR, ZR)])
        return 0

    lax.fori_loop(0, ZPT, wbody, 0)


def _scat_call(hs, src3, dst3):
    return pl.kernel(
        _scat_body,
        out_type=jax.ShapeDtypeStruct((NC, N, D), jnp.float32),
        mesh=_mesh(),
        scratch_types=[
            pltpu.VMEM_SHARED((N, D), jnp.float32),
            pltpu.VMEM((NCHUNK, CH), jnp.int32),
            pltpu.VMEM((NCHUNK, CH), jnp.int32),
            pltpu.VMEM((CH, D), jnp.float32),
            pltpu.VMEM((ZR, D), jnp.float32),
            pltpu.SemaphoreType.DMA,
        ],
        compiler_params=pltpu.CompilerParams(needs_layout_passes=False),
    )(hs, src3, dst3)


BR = 2000


def _pre_body(x_ref, w_ref, degp_ref, hs_ref):
    deg = jnp.sum(degp_ref[...], axis=1).astype(jnp.float32) + 1.0
    dinv = lax.rsqrt(deg)
    h = jnp.dot(x_ref[...], w_ref[...], preferred_element_type=jnp.float32)
    hs_ref[...] = h * dinv[:, None]


def _pre_call(x, W, degp_t):
    grid = (N // BR,)
    return pl.pallas_call(
        _pre_body,
        grid=grid,
        in_specs=[
            pl.BlockSpec((BR, D), lambda i: (i, 0)),
            pl.BlockSpec((D, D), lambda i: (0, 0)),
            pl.BlockSpec((BR, NW), lambda i: (i, 0)),
        ],
        out_specs=pl.BlockSpec((BR, D), lambda i: (i, 0)),
        out_shape=jax.ShapeDtypeStruct((N, D), jnp.float32),
    )(x, W, degp_t)


def _post_body(s01_ref, hs_ref, x_ref, degp_ref, b_ref, g_ref, bt_ref,
               out_ref):
    deg = jnp.sum(degp_ref[...], axis=1).astype(jnp.float32) + 1.0
    dinv = lax.rsqrt(deg)
    ssum = s01_ref[0] + s01_ref[1]
    pre = dinv[:, None] * (ssum + hs_ref[...]) + b_ref[...]
    h = jnp.maximum(pre, 0.0) + x_ref[...]
    mean = jnp.mean(h, axis=-1, keepdims=True)
    hc = h - mean
    var = jnp.mean(hc * hc, axis=-1, keepdims=True)
    out_ref[...] = hc * lax.rsqrt(var + EPS) * g_ref[...] + bt_ref[...]


def _post_call(s01, hs, x, degp_t, b, gamma, beta):
    grid = (N // BR,)
    return pl.pallas_call(
        _post_body,
        grid=grid,
        in_specs=[
            pl.BlockSpec((2, BR, D), lambda i: (0, i, 0)),
            pl.BlockSpec((BR, D), lambda i: (i, 0)),
            pl.BlockSpec((BR, D), lambda i: (i, 0)),
            pl.BlockSpec((BR, NW), lambda i: (i, 0)),
            pl.BlockSpec((1, D), lambda i: (0, 0)),
            pl.BlockSpec((1, D), lambda i: (0, 0)),
            pl.BlockSpec((1, D), lambda i: (0, 0)),
        ],
        out_specs=pl.BlockSpec((BR, D), lambda i: (i, 0)),
        out_shape=jax.ShapeDtypeStruct((N, D), jnp.float32),
    )(s01, hs, x, degp_t, b[None, :], gamma[None, :], beta[None, :])


def kernel(x, edge_index, W, b, gamma, beta):
    src = edge_index[0].astype(jnp.int32)
    dst = edge_index[1].astype(jnp.int32)
    src3 = src.reshape(NW, NCHUNK, CH)
    dst3 = dst.reshape(NW, NCHUNK, CH)

    degp_t = jnp.transpose(_deg_call(dst))
    hs = _pre_call(x, W, degp_t)
    s01 = _scat_call(hs, src3, dst3)
    return _post_call(s01, hs, x, degp_t, b, gamma, beta)
